# Initial kernel scaffold; baseline (speedup 1.0000x reference)
import jax
import jax.numpy as jnp
from jax.experimental import pallas as pl


def kernel(node_rep, edge_rep, edge_attr, edge_index, W1, W2, W3, We, bn_g, bn_b, M1, g1, b1, M2, g2, b2, M3, bias3):
    raise NotImplementedError("write your pallas kernel here")


# speedup vs baseline: 1.8525x; 1.8525x over previous
import functools

import jax
import jax.numpy as jnp
from jax import lax
from jax.experimental import pallas as pl
from jax.experimental.pallas import tpu as pltpu
from jax.experimental.pallas import tpu_sc as plsc

_NC = 2
_NS = 16
_NW = _NC * _NS

_N = 10000
_E = 320000
_D = 128
_DE = 16
_H = 256

_EPT = _E // _NW
_KG = 80
_NCH = _EPT // _KG
_NWR = 10
_RPT = _N // _NWR
_EPS = 1e-5


def _mesh():
    return plsc.VectorSubcoreMesh(
        core_axis_name="c", subcore_axis_name="s",
        num_cores=_NC, num_subcores=_NS)


def _ab_body(x_ref, w1_ref, w2_ref, a_ref, b_ref):
    x = x_ref[...]
    a_ref[...] = jnp.dot(x, w1_ref[...], preferred_element_type=jnp.float32)
    b_ref[...] = jnp.dot(x, w2_ref[...], preferred_element_type=jnp.float32)


def _gather_body(a_hbm, b_hbm, src_hbm, dst_hbm, g_hbm,
                 idx_s, idx_d, rows, sem):
    cid = lax.axis_index("c")
    sid = lax.axis_index("s")
    wid = sid * _NC + cid
    base0 = wid * _EPT

    def chunk(j, carry):
        base = pl.multiple_of(base0 + j * _KG, 8)
        pltpu.sync_copy(src_hbm.at[pl.ds(base, _KG)], idx_s)
        pltpu.sync_copy(dst_hbm.at[pl.ds(base, _KG)], idx_d)
        pltpu.async_copy(a_hbm.at[idx_s], rows, sem).wait()
        pltpu.async_copy(b_hbm.at[idx_d], rows, sem, add=True).wait()
        pltpu.sync_copy(rows, g_hbm.at[pl.ds(base, _KG)])
        return carry

    lax.fori_loop(0, _NCH, chunk, 0)


def _msg_body(er_ref, ea_ref, g_ref, w3_ref, we_ref, m_ref, stats_ref):
    i = pl.program_id(0)
    m = jnp.dot(er_ref[...], w3_ref[...], preferred_element_type=jnp.float32)
    m = m + jnp.dot(ea_ref[...], we_ref[...],
                    preferred_element_type=jnp.float32)
    m = m + g_ref[...]
    m_ref[...] = m
    part = jnp.stack([jnp.sum(m, axis=0), jnp.sum(m * m, axis=0)])

    @pl.when(i == 0)
    def _():
        stats_ref[...] = jnp.zeros_like(stats_ref)

    stats_ref[...] += part


def _scatter_body(m_hbm, dst_hbm, ab_hbm, zeros_hbm, y_hbm,
                  aff, dst_v, rows, ysh, sem):
    cid = lax.axis_index("c")
    sid = lax.axis_index("s")
    wid = sid * _NC + cid
    row0 = pl.multiple_of(sid * _RPT, 8)

    @pl.when(sid < _NWR)
    def _():
        pltpu.sync_copy(zeros_hbm.at[pl.ds(row0, _RPT)],
                        ysh.at[pl.ds(row0, _RPT)])

    pltpu.sync_copy(ab_hbm, aff)
    plsc.subcore_barrier()

    base0 = wid * _EPT

    def chunk(j, carry):
        base = pl.multiple_of(base0 + j * _KG, 8)
        pltpu.sync_copy(dst_hbm.at[pl.ds(base, _KG)], dst_v)
        pltpu.sync_copy(m_hbm.at[pl.ds(base, _KG)], rows)

        def row(r, c2):
            for u in range(_D // 16):
                sl = pl.ds(u * 16, 16)
                v = rows[r, sl] * aff[0, sl] + aff[1, sl]
                rows[r, sl] = jnp.maximum(v, 0.0)
            return c2

        lax.fori_loop(0, _KG, row, 0)
        pltpu.sync_copy(rows, ysh.at[dst_v], add=True)
        return carry

    lax.fori_loop(0, _NCH, chunk, 0)
    plsc.subcore_barrier()

    @pl.when(sid < _NWR)
    def _():
        pltpu.sync_copy(ysh.at[pl.ds(row0, _RPT)],
                        y_hbm.at[cid, pl.ds(row0, _RPT)])


def _mlp1_body(y2_ref, m1_ref, h_ref, stats_ref):
    i = pl.program_id(0)
    y = y2_ref[0] + y2_ref[1]
    h = jnp.dot(y, m1_ref[...], preferred_element_type=jnp.float32)
    h_ref[...] = h
    part = jnp.stack([jnp.sum(h, axis=0), jnp.sum(h * h, axis=0)])

    @pl.when(i == 0)
    def _():
        stats_ref[...] = jnp.zeros_like(stats_ref)

    stats_ref[...] += part


def _mlp2_body(h_ref, aff_ref, m2_ref, o_ref, stats_ref):
    i = pl.program_id(0)
    x = jnp.maximum(h_ref[...] * aff_ref[0] + aff_ref[1], 0.0)
    o = jnp.dot(x, m2_ref[...], preferred_element_type=jnp.float32)
    o_ref[...] = o
    part = jnp.stack([jnp.sum(o, axis=0), jnp.sum(o * o, axis=0)])

    @pl.when(i == 0)
    def _():
        stats_ref[...] = jnp.zeros_like(stats_ref)

    stats_ref[...] += part


def _mlp3_body(h_ref, aff_ref, m3_ref, b3_ref, o_ref):
    x = jnp.maximum(h_ref[...] * aff_ref[0] + aff_ref[1], 0.0)
    o_ref[...] = jnp.dot(x, m3_ref[...],
                         preferred_element_type=jnp.float32) + b3_ref[...]


def _affine(stats, count, gamma, beta):
    mean = stats[0] / count
    var = stats[1] / count - mean * mean
    alpha = gamma * jax.lax.rsqrt(var + _EPS)
    return jnp.stack([alpha, beta - mean * alpha])


def kernel(node_rep, edge_rep, edge_attr, edge_index, W1, W2, W3, We,
           bn_g, bn_b, M1, g1, b1, M2, g2, b2, M3, bias3):
    src = edge_index[0].astype(jnp.int32)
    dst = edge_index[1].astype(jnp.int32)

    a, b = pl.pallas_call(
        _ab_body,
        out_shape=[jax.ShapeDtypeStruct((_N, _D), jnp.float32)] * 2,
    )(node_rep, W1, W2)

    g = pl.kernel(
        _gather_body,
        out_type=jax.ShapeDtypeStruct((_E, _D), jnp.float32),
        mesh=_mesh(),
        scratch_types=[
            pltpu.VMEM((_KG,), jnp.int32),
            pltpu.VMEM((_KG,), jnp.int32),
            pltpu.VMEM((_KG, _D), jnp.float32),
            pltpu.SemaphoreType.DMA,
        ],
    )(a, b, src, dst)

    be = 2000
    m, stats = pl.pallas_call(
        _msg_body,
        grid=(_E // be,),
        in_specs=[
            pl.BlockSpec((be, _D), lambda i: (i, 0)),
            pl.BlockSpec((be, _DE), lambda i: (i, 0)),
            pl.BlockSpec((be, _D), lambda i: (i, 0)),
            pl.BlockSpec((_D, _D), lambda i: (0, 0)),
            pl.BlockSpec((_DE, _D), lambda i: (0, 0)),
        ],
        out_specs=[
            pl.BlockSpec((be, _D), lambda i: (i, 0)),
            pl.BlockSpec((2, _D), lambda i: (0, 0)),
        ],
        out_shape=[
            jax.ShapeDtypeStruct((_E, _D), jnp.float32),
            jax.ShapeDtypeStruct((2, _D), jnp.float32),
        ],
    )(edge_rep, edge_attr, g, W3, We)

    ab_edge = _affine(stats, float(_E), bn_g, bn_b)
    zeros_n = jnp.zeros((_N, _D), jnp.float32)

    y2 = pl.kernel(
        _scatter_body,
        out_type=jax.ShapeDtypeStruct((_NC, _N, _D), jnp.float32),
        mesh=_mesh(),
        scratch_types=[
            pltpu.VMEM((2, _D), jnp.float32),
            pltpu.VMEM((_KG,), jnp.int32),
            pltpu.VMEM((_KG, _D), jnp.float32),
            pltpu.VMEM_SHARED((_N, _D), jnp.float32),
            pltpu.SemaphoreType.DMA,
        ],
    )(m, dst, ab_edge, zeros_n)

    br = 2000
    h1, s1 = pl.pallas_call(
        _mlp1_body,
        grid=(_N // br,),
        in_specs=[
            pl.BlockSpec((_NC, br, _D), lambda i: (0, i, 0)),
            pl.BlockSpec((_D, _H), lambda i: (0, 0)),
        ],
        out_specs=[
            pl.BlockSpec((br, _H), lambda i: (i, 0)),
            pl.BlockSpec((2, _H), lambda i: (0, 0)),
        ],
        out_shape=[
            jax.ShapeDtypeStruct((_N, _H), jnp.float32),
            jax.ShapeDtypeStruct((2, _H), jnp.float32),
        ],
    )(y2, M1)

    aff1 = _affine(s1, float(_N), g1, b1)
    h2, s2 = pl.pallas_call(
        _mlp2_body,
        grid=(_N // br,),
        in_specs=[
            pl.BlockSpec((br, _H), lambda i: (i, 0)),
            pl.BlockSpec((2, _H), lambda i: (0, 0)),
            pl.BlockSpec((_H, _H), lambda i: (0, 0)),
        ],
        out_specs=[
            pl.BlockSpec((br, _H), lambda i: (i, 0)),
            pl.BlockSpec((2, _H), lambda i: (0, 0)),
        ],
        out_shape=[
            jax.ShapeDtypeStruct((_N, _H), jnp.float32),
            jax.ShapeDtypeStruct((2, _H), jnp.float32),
        ],
    )(h1, aff1, M2)

    aff2 = _affine(s2, float(_N), g2, b2)
    out = pl.pallas_call(
        _mlp3_body,
        grid=(_N // br,),
        in_specs=[
            pl.BlockSpec((br, _H), lambda i: (i, 0)),
            pl.BlockSpec((2, _H), lambda i: (0, 0)),
            pl.BlockSpec((_H, _D), lambda i: (0, 0)),
            pl.BlockSpec((1, _D), lambda i: (0, 0)),
        ],
        out_specs=pl.BlockSpec((br, _D), lambda i: (i, 0)),
        out_shape=jax.ShapeDtypeStruct((_N, _D), jnp.float32),
    )(h2, aff2, M3, bias3.reshape(1, _D))

    return out
